# baseline (speedup 1.0000x reference)
import functools

import jax
import jax.numpy as jnp
from jax import lax
from jax.experimental import pallas as pl
from jax.experimental.pallas import tpu as pltpu
from jax.experimental.pallas import tpu_sc as plsc

N = 10000
NP = 10240
E = 320000
IN_C = 128
HID = 256
OUT_C = 64
K = 10

NT = 32
NSUB = 16
DEG_B = 79
HOP_B = 84
ED = NT * DEG_B * 128
EH = NT * HOP_B * 128
RPT = NP // NSUB
RPW = NP // NT
NBUF = 4

_mesh = plsc.VectorSubcoreMesh(core_axis_name="c", subcore_axis_name="s")
_sc_params = pltpu.CompilerParams(needs_layout_passes=False,
                                  use_tc_tiling_on_sc=False)


def _rsqrt16(d):
    i = plsc.bitcast(d, jnp.int32)
    y = plsc.bitcast(jnp.int32(0x5F3759DF) - (i >> 1), jnp.float32)
    for _ in range(3):
        y = y * (1.5 - 0.5 * d * y * y)
    return y


@functools.partial(
    pl.kernel,
    mesh=_mesh,
    out_type=jax.ShapeDtypeStruct((NT, NP), jnp.float32),
    scratch_types=[
        pltpu.VMEM((DEG_B * 128,), jnp.int32),
        pltpu.VMEM((DEG_B * 128,), jnp.int32),
        pltpu.VMEM((NP,), jnp.float32),
    ],
    compiler_params=_sc_params,
)
def _deg_kernel(rowd, cold, part, rvm, cvm, acc):
    c = lax.axis_index("c")
    s = lax.axis_index("s")
    wid = c * NSUB + s
    pltpu.sync_copy(rowd.at[wid], rvm)
    pltpu.sync_copy(cold.at[wid], cvm)

    def _zero(i, carry):
        acc[pl.ds(i * 16, 16)] = jnp.zeros((16,), jnp.float32)
        return carry

    lax.fori_loop(0, NP // 16, _zero, 0)

    def _edge(i, carry):
        r = rvm[pl.ds(i * 16, 16)]
        cc = cvm[pl.ds(i * 16, 16)]
        ew = jnp.where(r != cc, 1.0, 0.0).astype(jnp.float32)
        plsc.addupdate_scatter(acc, [cc], ew)
        return carry

    lax.fori_loop(0, (DEG_B * 128) // 16, _edge, 0)
    pltpu.sync_copy(acc, part.at[wid])


@functools.partial(
    pl.kernel,
    mesh=_mesh,
    out_type=[
        jax.ShapeDtypeStruct((NP,), jnp.float32),
        jax.ShapeDtypeStruct((2, NP, OUT_C), jnp.float32),
    ],
    scratch_types=[
        pltpu.VMEM((NT, RPW), jnp.float32),
        pltpu.VMEM((RPW,), jnp.float32),
        pltpu.VMEM((RPW,), jnp.float32),
        pltpu.VMEM((RPW, OUT_C), jnp.float32),
        pltpu.VMEM((RPW, OUT_C), jnp.float32),
    ],
    compiler_params=_sc_params,
)
def _prep_kernel(part, h, dis_out, pinit, pbuf, disv, sdv, hv, wv):
    c = lax.axis_index("c")
    s = lax.axis_index("s")
    wid = c * NSUB + s
    base = wid * RPW
    for k in range(NT):
        pltpu.sync_copy(part.at[k, pl.ds(base, RPW)], pbuf.at[k])
    pltpu.sync_copy(h.at[pl.ds(base, RPW)], hv)

    def _vec(v, carry):
        d = pbuf[0, pl.ds(v * 16, 16)]
        for k in range(1, NT):
            d = d + pbuf[k, pl.ds(v * 16, 16)]
        d = d + 1.0
        r = _rsqrt16(d)
        disv[pl.ds(v * 16, 16)] = r
        sdv[pl.ds(v * 16, 16)] = d * r
        return carry

    lax.fori_loop(0, RPW // 16, _vec, 0)
    pltpu.sync_copy(disv, dis_out.at[pl.ds(base, RPW)])

    def _rowg(g, carry):
        s16 = sdv[pl.ds(g * 16, 16)]
        i16 = base + g * 16 + lax.iota(jnp.int32, 16)
        s16 = jnp.where(i16 < N, s16, 0.0)
        for r in range(16):
            i = g * 16 + r
            sc = s16[r]
            for l in range(OUT_C // 16):
                wv[i, pl.ds(l * 16, 16)] = hv[i, pl.ds(l * 16, 16)] * sc
        return carry

    lax.fori_loop(0, RPW // 16, _rowg, 0)
    pltpu.sync_copy(wv, pinit.at[0, pl.ds(base, RPW)])

    def _zrow(i, carry):
        for l in range(OUT_C // 16):
            wv[i, pl.ds(l * 16, 16)] = jnp.zeros((16,), jnp.float32)
        return carry

    lax.fori_loop(0, RPW, _zrow, 0)
    pltpu.sync_copy(wv, pinit.at[1, pl.ds(base, RPW)])


@functools.partial(
    pl.kernel,
    mesh=_mesh,
    out_type=[
        jax.ShapeDtypeStruct((2, NP, OUT_C), jnp.float32),
        jax.ShapeDtypeStruct((NP, OUT_C), jnp.float32),
        jax.ShapeDtypeStruct((2 * NP, OUT_C), jnp.float32),
    ],
    scratch_types=[
        pltpu.VMEM((HOP_B, 128), jnp.int32),
        pltpu.VMEM((HOP_B, 128), jnp.int32),
        pltpu.VMEM((RPT,), jnp.float32),
        pltpu.VMEM((128, OUT_C), jnp.float32),
        pltpu.VMEM((128, OUT_C), jnp.float32),
        [pltpu.VMEM((128, OUT_C), jnp.float32) for _ in range(2)],
        [pltpu.SemaphoreType.DMA for _ in range(2)],
        [pltpu.SemaphoreType.DMA for _ in range(2)],
        pltpu.VMEM_SHARED((NP, OUT_C), jnp.float32),
    ],
    compiler_params=_sc_params,
)
def _hop_kernel(pprev, dis, rowh2, colh, part, pred, u_out,
                rvm, cvm, disv, pa, pb, gb, sg, ss, acc):
    c = lax.axis_index("c")
    s = lax.axis_index("s")
    wid = c * NSUB + s
    pltpu.sync_copy(rowh2.at[c, wid], rvm)
    pltpu.sync_copy(colh.at[wid], cvm)
    base = s * RPT
    pltpu.sync_copy(dis.at[pl.ds(base, RPT)], disv)

    scope1 = jax.named_scope("hop_phase1")
    scope1.__enter__()
    for sub in range(RPT // 128):
        rb = base + sub * 128
        pltpu.sync_copy(pprev.at[0, pl.ds(rb, 128)], pa)
        pltpu.sync_copy(pprev.at[1, pl.ds(rb, 128)], pb)

        def _rowg(g, carry, _sub=sub):
            d16 = disv[pl.ds(_sub * 128 + g * 16, 16)]
            for r in range(16):
                i = g * 16 + r
                d = d16[r]
                for l in range(OUT_C // 16):
                    t = (pa[i, pl.ds(l * 16, 16)]
                         + pb[i, pl.ds(l * 16, 16)]) * d
                    pa[i, pl.ds(l * 16, 16)] = t
                    pb[i, pl.ds(l * 16, 16)] = t * d
            return carry

        lax.fori_loop(0, 8, _rowg, 0)

        @pl.when(c == 0)
        def _():
            pltpu.sync_copy(pa, pred.at[pl.ds(rb, 128)])

        pltpu.sync_copy(pb, u_out.at[pl.ds(c * NP + rb, 128)])

        def _zrow(i, carry):
            for l in range(OUT_C // 16):
                pa[i, pl.ds(l * 16, 16)] = jnp.zeros((16,), jnp.float32)
            return carry

        lax.fori_loop(0, 128, _zrow, 0)
        pltpu.sync_copy(pa, acc.at[pl.ds(rb, 128)])

    scope1.__exit__(None, None, None)
    plsc.subcore_barrier()

    scope2 = jax.named_scope("hop_phase2")
    scope2.__enter__()

    def _blk(j, carry):
        pltpu.async_copy(u_out.at[rvm.at[j]], gb[0], sg[0]).wait()
        pltpu.sync_copy(gb[0], acc.at[cvm.at[j]], add=True)
        return carry

    lax.fori_loop(0, HOP_B, _blk, 0)
    scope2.__exit__(None, None, None)

    plsc.subcore_barrier()
    pltpu.sync_copy(acc.at[pl.ds(base, RPT)], part.at[c, pl.ds(base, RPT)])


@functools.partial(
    pl.kernel,
    mesh=_mesh,
    out_type=jax.ShapeDtypeStruct((NP, OUT_C), jnp.float32),
    scratch_types=[
        pltpu.VMEM((RPW,), jnp.float32),
        pltpu.VMEM((RPW, OUT_C), jnp.float32),
        pltpu.VMEM((RPW, OUT_C), jnp.float32),
    ],
    compiler_params=_sc_params,
)
def _merge_kernel(pprev, dis, pred, disv, pa, pb):
    c = lax.axis_index("c")
    s = lax.axis_index("s")
    wid = c * NSUB + s
    base = wid * RPW
    pltpu.sync_copy(dis.at[pl.ds(base, RPW)], disv)
    pltpu.sync_copy(pprev.at[0, pl.ds(base, RPW)], pa)
    pltpu.sync_copy(pprev.at[1, pl.ds(base, RPW)], pb)

    def _rowg(g, carry):
        d16 = disv[pl.ds(g * 16, 16)]
        for r in range(16):
            i = g * 16 + r
            d = d16[r]
            for l in range(OUT_C // 16):
                pa[i, pl.ds(l * 16, 16)] = (
                    pa[i, pl.ds(l * 16, 16)] + pb[i, pl.ds(l * 16, 16)]) * d
        return carry

    lax.fori_loop(0, RPW // 16, _rowg, 0)
    pltpu.sync_copy(pa, pred.at[pl.ds(base, RPW)])


def _mlp_body(x_ref, w1_ref, b1_ref, w2_ref, b2_ref, h_ref):
    h1 = jnp.maximum(
        jnp.dot(x_ref[...], w1_ref[...], preferred_element_type=jnp.float32)
        + b1_ref[...], 0.0)
    h_ref[...] = (
        jnp.dot(h1, w2_ref[...], preferred_element_type=jnp.float32)
        + b2_ref[...])


MLP_BLK = 512


def _mlp(x_pad, W1, b1, W2, b2):
    return pl.pallas_call(
        _mlp_body,
        grid=(NP // MLP_BLK,),
        in_specs=[
            pl.BlockSpec((MLP_BLK, IN_C), lambda i: (i, 0)),
            pl.BlockSpec((IN_C, HID), lambda i: (0, 0)),
            pl.BlockSpec((1, HID), lambda i: (0, 0)),
            pl.BlockSpec((HID, OUT_C), lambda i: (0, 0)),
            pl.BlockSpec((1, OUT_C), lambda i: (0, 0)),
        ],
        out_specs=pl.BlockSpec((MLP_BLK, OUT_C), lambda i: (i, 0)),
        out_shape=jax.ShapeDtypeStruct((NP, OUT_C), jnp.float32),
    )(x_pad, W1, b1.reshape(1, HID), W2, b2.reshape(1, OUT_C))


CB = 128


def _final_body(*refs):
    pred_refs = refs[:K + 1]
    wp_ref, bp_ref, out_ref = refs[K + 1], refs[K + 2], refs[K + 3]
    acc = jnp.zeros((CB, OUT_C), jnp.float32)
    for pr in pred_refs:
        p = pr[...]
        r = jax.nn.sigmoid(
            jnp.dot(p, wp_ref[...], preferred_element_type=jnp.float32)
            + bp_ref[...])
        acc = acc + r * p
    out_ref[...] = acc


def _final(preds, Wp, bp2):
    return pl.pallas_call(
        _final_body,
        grid=(NP // CB,),
        in_specs=(
            [pl.BlockSpec((CB, OUT_C), lambda i: (i, 0)) for _ in range(K + 1)]
            + [pl.BlockSpec((OUT_C, 1), lambda i: (0, 0)),
               pl.BlockSpec((1, 1), lambda i: (0, 0))]
        ),
        out_specs=pl.BlockSpec((CB, OUT_C), lambda i: (i, 0)),
        out_shape=jax.ShapeDtypeStruct((NP, OUT_C), jnp.float32),
    )(*preds, Wp, bp2)


def _interleave(flat):
    return flat.reshape(NT, HOP_B, 128)


def kernel(x, edge_index, W1, b1, W2, b2, Wp, bp):
    row = edge_index[0]
    col = edge_index[1]
    loop = jnp.arange(N, dtype=jnp.int32)

    padD = jnp.full((ED - E,), N, dtype=jnp.int32)
    rowd = jnp.concatenate([row, padD]).reshape(NT, DEG_B * 128)
    cold = jnp.concatenate([col, padD]).reshape(NT, DEG_B * 128)

    rowp = jnp.where(row == col, N, row)
    padH = jnp.full((EH - E - N,), N, dtype=jnp.int32)
    rowh = _interleave(jnp.concatenate([rowp, loop, padH]))
    colh = _interleave(jnp.concatenate([col, loop, padH]))
    rowh2 = jnp.stack([rowh, rowh + NP])

    x_pad = jnp.concatenate([x, jnp.zeros((NP - N, IN_C), jnp.float32)])
    bp2 = bp.reshape(1, 1)

    h = _mlp(x_pad, W1, b1, W2, b2)
    part_deg = _deg_kernel(rowd, cold)
    dis, p = _prep_kernel(part_deg, h)
    preds = []
    for _ in range(K):
        p, pk, _u = _hop_kernel(p, dis, rowh2, colh)
        preds.append(pk)
    preds.append(_merge_kernel(p, dis))
    out = _final(preds, Wp, bp2)
    return out[:N]


# speedup vs baseline: 1.9837x; 1.9837x over previous
import functools

import jax
import jax.numpy as jnp
from jax import lax
from jax.experimental import pallas as pl
from jax.experimental.pallas import tpu as pltpu
from jax.experimental.pallas import tpu_sc as plsc

N = 10000
NP = 10240
E = 320000
IN_C = 128
HID = 256
OUT_C = 64
K = 10

NT = 32
NSUB = 16
DEG_B = 79
HOP_B = 84
ED = NT * DEG_B * 128
EH = NT * HOP_B * 128
RPT = NP // NSUB
RPW = NP // NT
NBUF = 6

_mesh = plsc.VectorSubcoreMesh(core_axis_name="c", subcore_axis_name="s")
_sc_params = pltpu.CompilerParams(needs_layout_passes=False,
                                  use_tc_tiling_on_sc=False)


def _rsqrt16(d):
    i = plsc.bitcast(d, jnp.int32)
    y = plsc.bitcast(jnp.int32(0x5F3759DF) - (i >> 1), jnp.float32)
    for _ in range(3):
        y = y * (1.5 - 0.5 * d * y * y)
    return y


@functools.partial(
    pl.kernel,
    mesh=_mesh,
    out_type=jax.ShapeDtypeStruct((NT, NP), jnp.float32),
    scratch_types=[
        pltpu.VMEM((DEG_B * 128,), jnp.int32),
        pltpu.VMEM((DEG_B * 128,), jnp.int32),
        pltpu.VMEM((NP,), jnp.float32),
    ],
    compiler_params=_sc_params,
)
def _deg_kernel(rowd, cold, part, rvm, cvm, acc):
    c = lax.axis_index("c")
    s = lax.axis_index("s")
    wid = c * NSUB + s
    pltpu.sync_copy(rowd.at[wid], rvm)
    pltpu.sync_copy(cold.at[wid], cvm)

    def _zero(i, carry):
        acc[pl.ds(i * 16, 16)] = jnp.zeros((16,), jnp.float32)
        return carry

    lax.fori_loop(0, NP // 16, _zero, 0)

    def _edge(i, carry):
        r = rvm[pl.ds(i * 16, 16)]
        cc = cvm[pl.ds(i * 16, 16)]
        ew = jnp.where(r != cc, 1.0, 0.0).astype(jnp.float32)
        plsc.addupdate_scatter(acc, [cc], ew)
        return carry

    lax.fori_loop(0, (DEG_B * 128) // 16, _edge, 0)
    pltpu.sync_copy(acc, part.at[wid])


@functools.partial(
    pl.kernel,
    mesh=_mesh,
    out_type=[
        jax.ShapeDtypeStruct((NP,), jnp.float32),
        jax.ShapeDtypeStruct((2, NP, OUT_C), jnp.float32),
    ],
    scratch_types=[
        pltpu.VMEM((NT, RPW), jnp.float32),
        pltpu.VMEM((RPW,), jnp.float32),
        pltpu.VMEM((RPW,), jnp.float32),
        pltpu.VMEM((RPW, OUT_C), jnp.float32),
        pltpu.VMEM((RPW, OUT_C), jnp.float32),
    ],
    compiler_params=_sc_params,
)
def _prep_kernel(part, h, dis_out, pinit, pbuf, disv, sdv, hv, wv):
    c = lax.axis_index("c")
    s = lax.axis_index("s")
    wid = c * NSUB + s
    base = wid * RPW
    for k in range(NT):
        pltpu.sync_copy(part.at[k, pl.ds(base, RPW)], pbuf.at[k])
    pltpu.sync_copy(h.at[pl.ds(base, RPW)], hv)

    def _vec(v, carry):
        d = pbuf[0, pl.ds(v * 16, 16)]
        for k in range(1, NT):
            d = d + pbuf[k, pl.ds(v * 16, 16)]
        d = d + 1.0
        r = _rsqrt16(d)
        disv[pl.ds(v * 16, 16)] = r
        sdv[pl.ds(v * 16, 16)] = d * r
        return carry

    lax.fori_loop(0, RPW // 16, _vec, 0)
    pltpu.sync_copy(disv, dis_out.at[pl.ds(base, RPW)])

    def _rowg(g, carry):
        s16 = sdv[pl.ds(g * 16, 16)]
        i16 = base + g * 16 + lax.iota(jnp.int32, 16)
        s16 = jnp.where(i16 < N, s16, 0.0)
        for r in range(16):
            i = g * 16 + r
            sc = s16[r]
            for l in range(OUT_C // 16):
                wv[i, pl.ds(l * 16, 16)] = hv[i, pl.ds(l * 16, 16)] * sc
        return carry

    lax.fori_loop(0, RPW // 16, _rowg, 0)
    pltpu.sync_copy(wv, pinit.at[0, pl.ds(base, RPW)])

    def _zrow(i, carry):
        for l in range(OUT_C // 16):
            wv[i, pl.ds(l * 16, 16)] = jnp.zeros((16,), jnp.float32)
        return carry

    lax.fori_loop(0, RPW, _zrow, 0)
    pltpu.sync_copy(wv, pinit.at[1, pl.ds(base, RPW)])


@functools.partial(
    pl.kernel,
    mesh=_mesh,
    out_type=[
        jax.ShapeDtypeStruct((2, NP, OUT_C), jnp.float32),
        jax.ShapeDtypeStruct((NP, OUT_C), jnp.float32),
        jax.ShapeDtypeStruct((2 * NP, OUT_C), jnp.float32),
    ],
    scratch_types=[
        pltpu.VMEM((HOP_B, 128), jnp.int32),
        pltpu.VMEM((HOP_B, 128), jnp.int32),
        pltpu.VMEM((RPT,), jnp.float32),
        pltpu.VMEM((128, OUT_C), jnp.float32),
        pltpu.VMEM((128, OUT_C), jnp.float32),
        [pltpu.VMEM((128, OUT_C), jnp.float32) for _ in range(NBUF)],
        [pltpu.SemaphoreType.DMA for _ in range(NBUF)],
        [pltpu.SemaphoreType.DMA for _ in range(NBUF)],
        pltpu.VMEM_SHARED((NP, OUT_C), jnp.float32),
    ],
    compiler_params=_sc_params,
)
def _hop_kernel(pprev, dis, rowh2, colh, part, pred, u_out,
                rvm, cvm, disv, pa, pb, gb, sg, ss, acc):
    c = lax.axis_index("c")
    s = lax.axis_index("s")
    wid = c * NSUB + s
    pltpu.sync_copy(rowh2.at[c, wid], rvm)
    pltpu.sync_copy(colh.at[wid], cvm)
    base = s * RPT
    pltpu.sync_copy(dis.at[pl.ds(base, RPT)], disv)

    scope1 = jax.named_scope("hop_phase1")
    scope1.__enter__()
    for sub in range(RPT // 128):
        rb = base + sub * 128
        pltpu.sync_copy(pprev.at[0, pl.ds(rb, 128)], pa)
        pltpu.sync_copy(pprev.at[1, pl.ds(rb, 128)], pb)

        def _rowg(g, carry, _sub=sub):
            d16 = disv[pl.ds(_sub * 128 + g * 16, 16)]
            for r in range(16):
                i = g * 16 + r
                d = d16[r]
                for l in range(OUT_C // 16):
                    t = (pa[i, pl.ds(l * 16, 16)]
                         + pb[i, pl.ds(l * 16, 16)]) * d
                    pa[i, pl.ds(l * 16, 16)] = t
                    pb[i, pl.ds(l * 16, 16)] = t * d
            return carry

        lax.fori_loop(0, 8, _rowg, 0)

        @pl.when(c == 0)
        def _():
            pltpu.sync_copy(pa, pred.at[pl.ds(rb, 128)])

        pltpu.sync_copy(pb, u_out.at[pl.ds(c * NP + rb, 128)])

        def _zrow(i, carry):
            for l in range(OUT_C // 16):
                pa[i, pl.ds(l * 16, 16)] = jnp.zeros((16,), jnp.float32)
            return carry

        lax.fori_loop(0, 128, _zrow, 0)
        pltpu.sync_copy(pa, acc.at[pl.ds(rb, 128)])

    scope1.__exit__(None, None, None)
    plsc.subcore_barrier()

    scope2 = jax.named_scope("hop_phase2")
    scope2.__enter__()
    for b in range(NBUF):
        pltpu.async_copy(u_out.at[rvm.at[b]], gb[b], sg[b])

    def _ring(it, carry):
        for b in range(NBUF):
            j = it * NBUF + b
            pltpu.make_async_copy(u_out.at[rvm.at[j]], gb[b], sg[b]).wait()
            pltpu.async_copy(gb[b], acc.at[cvm.at[j]], ss[b], add=True)
        for b in range(NBUF):
            nj = it * NBUF + b + NBUF

            @pl.when(nj < HOP_B)
            def _(b=b, nj=nj):
                pltpu.make_async_copy(gb[b], acc.at[cvm.at[nj]], ss[b]).wait()
                pltpu.async_copy(u_out.at[rvm.at[nj]], gb[b], sg[b])
        return carry

    lax.fori_loop(0, HOP_B // NBUF, _ring, 0)
    for b in range(NBUF):
        pltpu.make_async_copy(gb[b], acc.at[cvm.at[0]], ss[b]).wait()
    scope2.__exit__(None, None, None)

    plsc.subcore_barrier()
    pltpu.sync_copy(acc.at[pl.ds(base, RPT)], part.at[c, pl.ds(base, RPT)])


@functools.partial(
    pl.kernel,
    mesh=_mesh,
    out_type=jax.ShapeDtypeStruct((NP, OUT_C), jnp.float32),
    scratch_types=[
        pltpu.VMEM((RPW,), jnp.float32),
        pltpu.VMEM((RPW, OUT_C), jnp.float32),
        pltpu.VMEM((RPW, OUT_C), jnp.float32),
    ],
    compiler_params=_sc_params,
)
def _merge_kernel(pprev, dis, pred, disv, pa, pb):
    c = lax.axis_index("c")
    s = lax.axis_index("s")
    wid = c * NSUB + s
    base = wid * RPW
    pltpu.sync_copy(dis.at[pl.ds(base, RPW)], disv)
    pltpu.sync_copy(pprev.at[0, pl.ds(base, RPW)], pa)
    pltpu.sync_copy(pprev.at[1, pl.ds(base, RPW)], pb)

    def _rowg(g, carry):
        d16 = disv[pl.ds(g * 16, 16)]
        for r in range(16):
            i = g * 16 + r
            d = d16[r]
            for l in range(OUT_C // 16):
                pa[i, pl.ds(l * 16, 16)] = (
                    pa[i, pl.ds(l * 16, 16)] + pb[i, pl.ds(l * 16, 16)]) * d
        return carry

    lax.fori_loop(0, RPW // 16, _rowg, 0)
    pltpu.sync_copy(pa, pred.at[pl.ds(base, RPW)])


def _mlp_body(x_ref, w1_ref, b1_ref, w2_ref, b2_ref, h_ref):
    h1 = jnp.maximum(
        jnp.dot(x_ref[...], w1_ref[...], preferred_element_type=jnp.float32)
        + b1_ref[...], 0.0)
    h_ref[...] = (
        jnp.dot(h1, w2_ref[...], preferred_element_type=jnp.float32)
        + b2_ref[...])


MLP_BLK = 512


def _mlp(x_pad, W1, b1, W2, b2):
    return pl.pallas_call(
        _mlp_body,
        grid=(NP // MLP_BLK,),
        in_specs=[
            pl.BlockSpec((MLP_BLK, IN_C), lambda i: (i, 0)),
            pl.BlockSpec((IN_C, HID), lambda i: (0, 0)),
            pl.BlockSpec((1, HID), lambda i: (0, 0)),
            pl.BlockSpec((HID, OUT_C), lambda i: (0, 0)),
            pl.BlockSpec((1, OUT_C), lambda i: (0, 0)),
        ],
        out_specs=pl.BlockSpec((MLP_BLK, OUT_C), lambda i: (i, 0)),
        out_shape=jax.ShapeDtypeStruct((NP, OUT_C), jnp.float32),
    )(x_pad, W1, b1.reshape(1, HID), W2, b2.reshape(1, OUT_C))


CB = 128


def _final_body(*refs):
    pred_refs = refs[:K + 1]
    wp_ref, bp_ref, out_ref = refs[K + 1], refs[K + 2], refs[K + 3]
    acc = jnp.zeros((CB, OUT_C), jnp.float32)
    for pr in pred_refs:
        p = pr[...]
        r = jax.nn.sigmoid(
            jnp.dot(p, wp_ref[...], preferred_element_type=jnp.float32)
            + bp_ref[...])
        acc = acc + r * p
    out_ref[...] = acc


def _final(preds, Wp, bp2):
    return pl.pallas_call(
        _final_body,
        grid=(NP // CB,),
        in_specs=(
            [pl.BlockSpec((CB, OUT_C), lambda i: (i, 0)) for _ in range(K + 1)]
            + [pl.BlockSpec((OUT_C, 1), lambda i: (0, 0)),
               pl.BlockSpec((1, 1), lambda i: (0, 0))]
        ),
        out_specs=pl.BlockSpec((CB, OUT_C), lambda i: (i, 0)),
        out_shape=jax.ShapeDtypeStruct((NP, OUT_C), jnp.float32),
    )(*preds, Wp, bp2)


def _interleave(flat):
    return flat.reshape(HOP_B * 128, NT).T.reshape(NT, HOP_B, 128)


def kernel(x, edge_index, W1, b1, W2, b2, Wp, bp):
    row = edge_index[0]
    col = edge_index[1]
    loop = jnp.arange(N, dtype=jnp.int32)

    padD = jnp.full((ED - E,), N, dtype=jnp.int32)
    rowd = jnp.concatenate([row, padD]).reshape(NT, DEG_B * 128)
    cold = jnp.concatenate([col, padD]).reshape(NT, DEG_B * 128)

    rowp = jnp.where(row == col, N, row)
    padH = jnp.full((EH - E - N,), N, dtype=jnp.int32)
    rowh = _interleave(jnp.concatenate([rowp, loop, padH]))
    colh = _interleave(jnp.concatenate([col, loop, padH]))
    rowh2 = jnp.stack([rowh, rowh + NP])

    x_pad = jnp.concatenate([x, jnp.zeros((NP - N, IN_C), jnp.float32)])
    bp2 = bp.reshape(1, 1)

    h = _mlp(x_pad, W1, b1, W2, b2)
    part_deg = _deg_kernel(rowd, cold)
    dis, p = _prep_kernel(part_deg, h)
    preds = []
    for _ in range(K):
        p, pk, _u = _hop_kernel(p, dis, rowh2, colh)
        preds.append(pk)
    preds.append(_merge_kernel(p, dis))
    out = _final(preds, Wp, bp2)
    return out[:N]


# speedup vs baseline: 3.3639x; 1.6957x over previous
import functools

import jax
import jax.numpy as jnp
from jax import lax
from jax.experimental import pallas as pl
from jax.experimental.pallas import tpu as pltpu
from jax.experimental.pallas import tpu_sc as plsc

N = 10000
NP = 10240
E = 320000
IN_C = 128
HID = 256
OUT_C = 64
K = 10

NT = 32
NSUB = 16
DEG_B = 79
HOP_B = 84
ED = NT * DEG_B * 128
EH = NT * HOP_B * 128
RPT = NP // NSUB
RPW = NP // NT
NBUF = 2
P1B = 64

_mesh = plsc.VectorSubcoreMesh(core_axis_name="c", subcore_axis_name="s")
_sc_params = pltpu.CompilerParams(needs_layout_passes=False,
                                  use_tc_tiling_on_sc=False)


def _rsqrt16(d):
    i = plsc.bitcast(d, jnp.int32)
    y = plsc.bitcast(jnp.int32(0x5F3759DF) - (i >> 1), jnp.float32)
    for _ in range(3):
        y = y * (1.5 - 0.5 * d * y * y)
    return y


@functools.partial(
    pl.kernel,
    mesh=_mesh,
    out_type=jax.ShapeDtypeStruct((NT, NP), jnp.float32),
    scratch_types=[
        pltpu.VMEM((DEG_B * 128,), jnp.int32),
        pltpu.VMEM((DEG_B * 128,), jnp.int32),
        pltpu.VMEM((NP,), jnp.float32),
    ],
    compiler_params=_sc_params,
)
def _deg_kernel(rowd, cold, part, rvm, cvm, acc):
    c = lax.axis_index("c")
    s = lax.axis_index("s")
    wid = c * NSUB + s
    pltpu.sync_copy(rowd.at[wid], rvm)
    pltpu.sync_copy(cold.at[wid], cvm)

    def _zero(i, carry):
        acc[pl.ds(i * 16, 16)] = jnp.zeros((16,), jnp.float32)
        return carry

    lax.fori_loop(0, NP // 16, _zero, 0)

    def _edge(i, carry):
        r = rvm[pl.ds(i * 16, 16)]
        cc = cvm[pl.ds(i * 16, 16)]
        ew = jnp.where(r != cc, 1.0, 0.0).astype(jnp.float32)
        plsc.addupdate_scatter(acc, [cc], ew)
        return carry

    lax.fori_loop(0, (DEG_B * 128) // 16, _edge, 0)
    pltpu.sync_copy(acc, part.at[wid])


@functools.partial(
    pl.kernel,
    mesh=_mesh,
    out_type=[
        jax.ShapeDtypeStruct((NP,), jnp.float32),
        jax.ShapeDtypeStruct((2, NP, OUT_C), jnp.float32),
    ],
    scratch_types=[
        pltpu.VMEM((NT, RPW), jnp.float32),
        pltpu.VMEM((RPW,), jnp.float32),
        pltpu.VMEM((RPW,), jnp.float32),
        pltpu.VMEM((RPW, OUT_C), jnp.float32),
        pltpu.VMEM((RPW, OUT_C), jnp.float32),
    ],
    compiler_params=_sc_params,
)
def _prep_kernel(part, h, dis_out, pinit, pbuf, disv, sdv, hv, wv):
    c = lax.axis_index("c")
    s = lax.axis_index("s")
    wid = c * NSUB + s
    base = wid * RPW
    for k in range(NT):
        pltpu.sync_copy(part.at[k, pl.ds(base, RPW)], pbuf.at[k])
    pltpu.sync_copy(h.at[pl.ds(base, RPW)], hv)

    def _vec(v, carry):
        d = pbuf[0, pl.ds(v * 16, 16)]
        for k in range(1, NT):
            d = d + pbuf[k, pl.ds(v * 16, 16)]
        d = d + 1.0
        r = _rsqrt16(d)
        disv[pl.ds(v * 16, 16)] = r
        sdv[pl.ds(v * 16, 16)] = d * r
        return carry

    lax.fori_loop(0, RPW // 16, _vec, 0)
    pltpu.sync_copy(disv, dis_out.at[pl.ds(base, RPW)])

    def _rowg(g, carry):
        s16 = sdv[pl.ds(g * 16, 16)]
        i16 = base + g * 16 + lax.iota(jnp.int32, 16)
        s16 = jnp.where(i16 < N, s16, 0.0)
        for r in range(16):
            i = g * 16 + r
            sc = s16[r]
            for l in range(OUT_C // 16):
                wv[i, pl.ds(l * 16, 16)] = hv[i, pl.ds(l * 16, 16)] * sc
        return carry

    lax.fori_loop(0, RPW // 16, _rowg, 0)
    pltpu.sync_copy(wv, pinit.at[0, pl.ds(base, RPW)])

    def _zrow(i, carry):
        for l in range(OUT_C // 16):
            wv[i, pl.ds(l * 16, 16)] = jnp.zeros((16,), jnp.float32)
        return carry

    lax.fori_loop(0, RPW, _zrow, 0)
    pltpu.sync_copy(wv, pinit.at[1, pl.ds(base, RPW)])


@functools.partial(
    pl.kernel,
    mesh=_mesh,
    out_type=[
        jax.ShapeDtypeStruct((2, NP, OUT_C), jnp.float32),
        jax.ShapeDtypeStruct((NP, OUT_C), jnp.float32),
    ],
    scratch_types=[
        pltpu.VMEM((HOP_B, 128), jnp.int32),
        pltpu.VMEM((HOP_B, 128), jnp.int32),
        pltpu.VMEM((RPT,), jnp.float32),
        pltpu.VMEM((P1B, OUT_C), jnp.float32),
        pltpu.VMEM((P1B, OUT_C), jnp.float32),
        [pltpu.VMEM((128, OUT_C), jnp.float32) for _ in range(NBUF)],
        [pltpu.SemaphoreType.DMA for _ in range(NBUF)],
        [pltpu.SemaphoreType.DMA for _ in range(NBUF)],
        pltpu.VMEM_SHARED((NP, OUT_C), jnp.float32),
        pltpu.VMEM_SHARED((NP, OUT_C), jnp.float32),
    ],
    compiler_params=_sc_params,
)
def _hop_kernel(pprev, dis, eh, part, pred,
                rvm, cvm, disv, pa, pb, gb, sg, ss, acc, u_sp):
    c = lax.axis_index("c")
    s = lax.axis_index("s")
    wid = c * NSUB + s
    pltpu.sync_copy(eh.at[wid], rvm)
    base = s * RPT
    pltpu.sync_copy(dis.at[pl.ds(base, RPT)], disv)

    def _unpack(j, carry):
        for l in range(128 // 16):
            v = rvm[j, pl.ds(l * 16, 16)]
            cvm[j, pl.ds(l * 16, 16)] = lax.bitwise_and(v, 16383)
            rvm[j, pl.ds(l * 16, 16)] = lax.shift_right_logical(v, 14)
        return carry

    lax.fori_loop(0, HOP_B, _unpack, 0)

    for sub in range(RPT // P1B):
        rb = base + sub * P1B
        pltpu.sync_copy(pprev.at[0, pl.ds(rb, P1B)], pa)
        pltpu.sync_copy(pprev.at[1, pl.ds(rb, P1B)], pb)

        def _rowg(g, carry, _sub=sub):
            d16 = disv[pl.ds(_sub * P1B + g * 16, 16)]
            for r in range(16):
                i = g * 16 + r
                d = d16[r]
                for l in range(OUT_C // 16):
                    t = (pa[i, pl.ds(l * 16, 16)]
                         + pb[i, pl.ds(l * 16, 16)]) * d
                    pa[i, pl.ds(l * 16, 16)] = t
                    pb[i, pl.ds(l * 16, 16)] = t * d
            return carry

        lax.fori_loop(0, P1B // 16, _rowg, 0)

        @pl.when(c == 0)
        def _():
            pltpu.sync_copy(pa, pred.at[pl.ds(rb, P1B)])

        pltpu.sync_copy(pb, u_sp.at[pl.ds(rb, P1B)])

        def _zrow(i, carry):
            for l in range(OUT_C // 16):
                pa[i, pl.ds(l * 16, 16)] = jnp.zeros((16,), jnp.float32)
            return carry

        lax.fori_loop(0, P1B, _zrow, 0)
        pltpu.sync_copy(pa, acc.at[pl.ds(rb, P1B)])

    plsc.subcore_barrier()

    for b in range(NBUF):
        pltpu.async_copy(u_sp.at[rvm.at[b]], gb[b], sg[b])

    def _ring(it, carry):
        for b in range(NBUF):
            j = it * NBUF + b
            pltpu.make_async_copy(u_sp.at[rvm.at[j]], gb[b], sg[b]).wait()
            pltpu.async_copy(gb[b], acc.at[cvm.at[j]], ss[b], add=True)
        for b in range(NBUF):
            nj = it * NBUF + b + NBUF

            @pl.when(nj < HOP_B)
            def _(b=b, nj=nj):
                pltpu.make_async_copy(gb[b], acc.at[cvm.at[nj]], ss[b]).wait()
                pltpu.async_copy(u_sp.at[rvm.at[nj]], gb[b], sg[b])
        return carry

    lax.fori_loop(0, HOP_B // NBUF, _ring, 0)
    for b in range(NBUF):
        pltpu.make_async_copy(gb[b], acc.at[cvm.at[0]], ss[b]).wait()

    plsc.subcore_barrier()
    pltpu.sync_copy(acc.at[pl.ds(base, RPT)], part.at[c, pl.ds(base, RPT)])


@functools.partial(
    pl.kernel,
    mesh=_mesh,
    out_type=jax.ShapeDtypeStruct((NP, OUT_C), jnp.float32),
    scratch_types=[
        pltpu.VMEM((RPW,), jnp.float32),
        pltpu.VMEM((RPW, OUT_C), jnp.float32),
        pltpu.VMEM((RPW, OUT_C), jnp.float32),
    ],
    compiler_params=_sc_params,
)
def _merge_kernel(pprev, dis, pred, disv, pa, pb):
    c = lax.axis_index("c")
    s = lax.axis_index("s")
    wid = c * NSUB + s
    base = wid * RPW
    pltpu.sync_copy(dis.at[pl.ds(base, RPW)], disv)
    pltpu.sync_copy(pprev.at[0, pl.ds(base, RPW)], pa)
    pltpu.sync_copy(pprev.at[1, pl.ds(base, RPW)], pb)

    def _rowg(g, carry):
        d16 = disv[pl.ds(g * 16, 16)]
        for r in range(16):
            i = g * 16 + r
            d = d16[r]
            for l in range(OUT_C // 16):
                pa[i, pl.ds(l * 16, 16)] = (
                    pa[i, pl.ds(l * 16, 16)] + pb[i, pl.ds(l * 16, 16)]) * d
        return carry

    lax.fori_loop(0, RPW // 16, _rowg, 0)
    pltpu.sync_copy(pa, pred.at[pl.ds(base, RPW)])


def _mlp_body(x_ref, w1_ref, b1_ref, w2_ref, b2_ref, h_ref):
    h1 = jnp.maximum(
        jnp.dot(x_ref[...], w1_ref[...], preferred_element_type=jnp.float32)
        + b1_ref[...], 0.0)
    h_ref[...] = (
        jnp.dot(h1, w2_ref[...], preferred_element_type=jnp.float32)
        + b2_ref[...])


MLP_BLK = 512


def _mlp(x_pad, W1, b1, W2, b2):
    return pl.pallas_call(
        _mlp_body,
        grid=(NP // MLP_BLK,),
        in_specs=[
            pl.BlockSpec((MLP_BLK, IN_C), lambda i: (i, 0)),
            pl.BlockSpec((IN_C, HID), lambda i: (0, 0)),
            pl.BlockSpec((1, HID), lambda i: (0, 0)),
            pl.BlockSpec((HID, OUT_C), lambda i: (0, 0)),
            pl.BlockSpec((1, OUT_C), lambda i: (0, 0)),
        ],
        out_specs=pl.BlockSpec((MLP_BLK, OUT_C), lambda i: (i, 0)),
        out_shape=jax.ShapeDtypeStruct((NP, OUT_C), jnp.float32),
    )(x_pad, W1, b1.reshape(1, HID), W2, b2.reshape(1, OUT_C))


CB = 128


def _final_body(*refs):
    pred_refs = refs[:K + 1]
    wp_ref, bp_ref, out_ref = refs[K + 1], refs[K + 2], refs[K + 3]
    acc = jnp.zeros((CB, OUT_C), jnp.float32)
    for pr in pred_refs:
        p = pr[...]
        r = jax.nn.sigmoid(
            jnp.dot(p, wp_ref[...], preferred_element_type=jnp.float32)
            + bp_ref[...])
        acc = acc + r * p
    out_ref[...] = acc


def _final(preds, Wp, bp2):
    return pl.pallas_call(
        _final_body,
        grid=(NP // CB,),
        in_specs=(
            [pl.BlockSpec((CB, OUT_C), lambda i: (i, 0)) for _ in range(K + 1)]
            + [pl.BlockSpec((OUT_C, 1), lambda i: (0, 0)),
               pl.BlockSpec((1, 1), lambda i: (0, 0))]
        ),
        out_specs=pl.BlockSpec((CB, OUT_C), lambda i: (i, 0)),
        out_shape=jax.ShapeDtypeStruct((NP, OUT_C), jnp.float32),
    )(*preds, Wp, bp2)


def _interleave(flat):
    return flat.reshape(HOP_B * 128, NT).T.reshape(NT, HOP_B, 128)


def kernel(x, edge_index, W1, b1, W2, b2, Wp, bp):
    row = edge_index[0]
    col = edge_index[1]
    loop = jnp.arange(N, dtype=jnp.int32)

    padD = jnp.full((ED - E,), N, dtype=jnp.int32)
    rowd = jnp.concatenate([row, padD]).reshape(NT, DEG_B * 128)
    cold = jnp.concatenate([col, padD]).reshape(NT, DEG_B * 128)

    rowp = jnp.where(row == col, N, row)
    padH = jnp.full((EH - E - N,), N, dtype=jnp.int32)
    rowh = jnp.concatenate([rowp, loop, padH])
    colh = jnp.concatenate([col, loop, padH])
    ehp = _interleave((rowh << 14) | colh)

    x_pad = jnp.concatenate([x, jnp.zeros((NP - N, IN_C), jnp.float32)])
    bp2 = bp.reshape(1, 1)

    h = _mlp(x_pad, W1, b1, W2, b2)
    part_deg = _deg_kernel(rowd, cold)
    dis, p = _prep_kernel(part_deg, h)
    preds = []
    for _ in range(K):
        p, pk = _hop_kernel(p, dis, ehp)
        preds.append(pk)
    preds.append(_merge_kernel(p, dis))
    out = _final(preds, Wp, bp2)
    return out[:N]


# speedup vs baseline: 3.4532x; 1.0266x over previous
import functools

import jax
import jax.numpy as jnp
from jax import lax
from jax.experimental import pallas as pl
from jax.experimental.pallas import tpu as pltpu
from jax.experimental.pallas import tpu_sc as plsc

N = 10000
NP = 10240
E = 320000
IN_C = 128
HID = 256
OUT_C = 64
K = 10

NT = 32
NSUB = 16
DEG_B = 79
HOP_B = 82
ED = NT * DEG_B * 128
EH = NT * HOP_B * 128
RPT = NP // NSUB
RPW = NP // NT
NBUF = 2
P1B = 64

_mesh = plsc.VectorSubcoreMesh(core_axis_name="c", subcore_axis_name="s")
_sc_params = pltpu.CompilerParams(needs_layout_passes=False,
                                  use_tc_tiling_on_sc=False)


def _rsqrt16(d):
    i = plsc.bitcast(d, jnp.int32)
    y = plsc.bitcast(jnp.int32(0x5F3759DF) - (i >> 1), jnp.float32)
    for _ in range(3):
        y = y * (1.5 - 0.5 * d * y * y)
    return y


@functools.partial(
    pl.kernel,
    mesh=_mesh,
    out_type=jax.ShapeDtypeStruct((NT, NP), jnp.float32),
    scratch_types=[
        pltpu.VMEM((DEG_B * 128,), jnp.int32),
        pltpu.VMEM((DEG_B * 128,), jnp.int32),
        pltpu.VMEM((NP,), jnp.float32),
    ],
    compiler_params=_sc_params,
)
def _deg_kernel(rowd, cold, part, rvm, cvm, acc):
    c = lax.axis_index("c")
    s = lax.axis_index("s")
    wid = c * NSUB + s
    pltpu.sync_copy(rowd.at[wid], rvm)
    pltpu.sync_copy(cold.at[wid], cvm)

    def _zero(i, carry):
        acc[pl.ds(i * 16, 16)] = jnp.zeros((16,), jnp.float32)
        return carry

    lax.fori_loop(0, NP // 16, _zero, 0)

    def _edge(i, carry):
        r = rvm[pl.ds(i * 16, 16)]
        cc = cvm[pl.ds(i * 16, 16)]
        ew = jnp.where(r != cc, 1.0, 0.0).astype(jnp.float32)
        plsc.addupdate_scatter(acc, [cc], ew)
        return carry

    lax.fori_loop(0, (DEG_B * 128) // 16, _edge, 0)
    pltpu.sync_copy(acc, part.at[wid])


@functools.partial(
    pl.kernel,
    mesh=_mesh,
    out_type=[
        jax.ShapeDtypeStruct((NP,), jnp.float32),
        jax.ShapeDtypeStruct((2, NP, OUT_C), jnp.float32),
    ],
    scratch_types=[
        pltpu.VMEM((NT, RPW), jnp.float32),
        pltpu.VMEM((RPW,), jnp.float32),
        pltpu.VMEM((RPW,), jnp.float32),
        pltpu.VMEM((RPW, OUT_C), jnp.float32),
        pltpu.VMEM((RPW, OUT_C), jnp.float32),
    ],
    compiler_params=_sc_params,
)
def _prep_kernel(part, h, dis_out, pinit, pbuf, disv, sdv, hv, wv):
    c = lax.axis_index("c")
    s = lax.axis_index("s")
    wid = c * NSUB + s
    base = wid * RPW
    for k in range(NT):
        pltpu.sync_copy(part.at[k, pl.ds(base, RPW)], pbuf.at[k])
    pltpu.sync_copy(h.at[pl.ds(base, RPW)], hv)

    def _vec(v, carry):
        d = pbuf[0, pl.ds(v * 16, 16)]
        for k in range(1, NT):
            d = d + pbuf[k, pl.ds(v * 16, 16)]
        d = d + 1.0
        r = _rsqrt16(d)
        disv[pl.ds(v * 16, 16)] = r
        sdv[pl.ds(v * 16, 16)] = d * r
        return carry

    lax.fori_loop(0, RPW // 16, _vec, 0)
    pltpu.sync_copy(disv, dis_out.at[pl.ds(base, RPW)])

    def _rowg(g, carry):
        s16 = sdv[pl.ds(g * 16, 16)]
        i16 = base + g * 16 + lax.iota(jnp.int32, 16)
        s16 = jnp.where(i16 < N, s16, 0.0)
        for r in range(16):
            i = g * 16 + r
            sc = s16[r]
            for l in range(OUT_C // 16):
                wv[i, pl.ds(l * 16, 16)] = hv[i, pl.ds(l * 16, 16)] * sc
        return carry

    lax.fori_loop(0, RPW // 16, _rowg, 0)
    pltpu.sync_copy(wv, pinit.at[0, pl.ds(base, RPW)])

    def _zrow(i, carry):
        for l in range(OUT_C // 16):
            wv[i, pl.ds(l * 16, 16)] = jnp.zeros((16,), jnp.float32)
        return carry

    lax.fori_loop(0, RPW, _zrow, 0)
    pltpu.sync_copy(wv, pinit.at[1, pl.ds(base, RPW)])


@functools.partial(
    pl.kernel,
    mesh=_mesh,
    out_type=[
        jax.ShapeDtypeStruct((2, NP, OUT_C), jnp.float32),
        jax.ShapeDtypeStruct((NP, OUT_C), jnp.float32),
    ],
    scratch_types=[
        pltpu.VMEM((HOP_B, 128), jnp.int32),
        pltpu.VMEM((HOP_B, 128), jnp.int32),
        pltpu.VMEM((RPT,), jnp.float32),
        pltpu.VMEM((P1B, OUT_C), jnp.float32),
        pltpu.VMEM((P1B, OUT_C), jnp.float32),
        [pltpu.VMEM((128, OUT_C), jnp.float32) for _ in range(NBUF)],
        [pltpu.SemaphoreType.DMA for _ in range(NBUF)],
        [pltpu.SemaphoreType.DMA for _ in range(NBUF)],
        pltpu.VMEM_SHARED((NP, OUT_C), jnp.float32),
        pltpu.VMEM_SHARED((NP, OUT_C), jnp.float32),
    ],
    compiler_params=_sc_params,
)
def _hop_kernel(pprev, dis, eh, part, pred,
                rvm, cvm, disv, pa, pb, gb, sg, ss, acc, u_sp):
    c = lax.axis_index("c")
    s = lax.axis_index("s")
    wid = c * NSUB + s
    pltpu.sync_copy(eh.at[wid], rvm)
    base = s * RPT
    pltpu.sync_copy(dis.at[pl.ds(base, RPT)], disv)

    def _unpack(j, carry):
        for l in range(128 // 16):
            v = rvm[j, pl.ds(l * 16, 16)]
            cvm[j, pl.ds(l * 16, 16)] = lax.bitwise_and(v, 16383)
            rvm[j, pl.ds(l * 16, 16)] = lax.shift_right_logical(v, 14)
        return carry

    lax.fori_loop(0, HOP_B, _unpack, 0)

    for sub in range(RPT // P1B):
        rb = base + sub * P1B
        pltpu.sync_copy(pprev.at[0, pl.ds(rb, P1B)], pa)
        pltpu.sync_copy(pprev.at[1, pl.ds(rb, P1B)], pb)

        def _rowg(g, carry, _sub=sub):
            d16 = disv[pl.ds(_sub * P1B + g * 16, 16)]
            for r in range(16):
                i = g * 16 + r
                d = d16[r]
                for l in range(OUT_C // 16):
                    t = (pa[i, pl.ds(l * 16, 16)]
                         + pb[i, pl.ds(l * 16, 16)]) * d
                    pa[i, pl.ds(l * 16, 16)] = t
                    pb[i, pl.ds(l * 16, 16)] = t * d
            return carry

        lax.fori_loop(0, P1B // 16, _rowg, 0)

        @pl.when(c == 0)
        def _():
            pltpu.sync_copy(pa, pred.at[pl.ds(rb, P1B)])

        pltpu.sync_copy(pb, u_sp.at[pl.ds(rb, P1B)])

        def _zrow(i, carry):
            for l in range(OUT_C // 16):
                pa[i, pl.ds(l * 16, 16)] = jnp.zeros((16,), jnp.float32)
            return carry

        lax.fori_loop(0, P1B, _zrow, 0)
        pltpu.sync_copy(pa, acc.at[pl.ds(rb, P1B)])

    plsc.subcore_barrier()

    for b in range(NBUF):
        pltpu.async_copy(u_sp.at[rvm.at[b]], gb[b], sg[b])

    def _ring(it, carry):
        for b in range(NBUF):
            j = it * NBUF + b
            pltpu.make_async_copy(u_sp.at[rvm.at[j]], gb[b], sg[b]).wait()
            pltpu.async_copy(gb[b], acc.at[cvm.at[j]], ss[b], add=True)
        for b in range(NBUF):
            nj = it * NBUF + b + NBUF

            @pl.when(nj < HOP_B)
            def _(b=b, nj=nj):
                pltpu.make_async_copy(gb[b], acc.at[cvm.at[nj]], ss[b]).wait()
                pltpu.async_copy(u_sp.at[rvm.at[nj]], gb[b], sg[b])
        return carry

    lax.fori_loop(0, HOP_B // NBUF, _ring, 0)
    for b in range(NBUF):
        pltpu.make_async_copy(gb[b], acc.at[cvm.at[0]], ss[b]).wait()

    plsc.subcore_barrier()
    pltpu.sync_copy(acc.at[pl.ds(base, RPT)], part.at[c, pl.ds(base, RPT)])


@functools.partial(
    pl.kernel,
    mesh=_mesh,
    out_type=jax.ShapeDtypeStruct((NP, OUT_C), jnp.float32),
    scratch_types=[
        pltpu.VMEM((RPW,), jnp.float32),
        pltpu.VMEM((RPW, OUT_C), jnp.float32),
        pltpu.VMEM((RPW, OUT_C), jnp.float32),
    ],
    compiler_params=_sc_params,
)
def _merge_kernel(pprev, dis, pred, disv, pa, pb):
    c = lax.axis_index("c")
    s = lax.axis_index("s")
    wid = c * NSUB + s
    base = wid * RPW
    pltpu.sync_copy(dis.at[pl.ds(base, RPW)], disv)
    pltpu.sync_copy(pprev.at[0, pl.ds(base, RPW)], pa)
    pltpu.sync_copy(pprev.at[1, pl.ds(base, RPW)], pb)

    def _rowg(g, carry):
        d16 = disv[pl.ds(g * 16, 16)]
        for r in range(16):
            i = g * 16 + r
            d = d16[r]
            for l in range(OUT_C // 16):
                pa[i, pl.ds(l * 16, 16)] = (
                    pa[i, pl.ds(l * 16, 16)] + pb[i, pl.ds(l * 16, 16)]) * d
        return carry

    lax.fori_loop(0, RPW // 16, _rowg, 0)
    pltpu.sync_copy(pa, pred.at[pl.ds(base, RPW)])


def _mlp_body(x_ref, w1_ref, b1_ref, w2_ref, b2_ref, h_ref):
    h1 = jnp.maximum(
        jnp.dot(x_ref[...], w1_ref[...], preferred_element_type=jnp.float32)
        + b1_ref[...], 0.0)
    h_ref[...] = (
        jnp.dot(h1, w2_ref[...], preferred_element_type=jnp.float32)
        + b2_ref[...])


MLP_BLK = 512


def _mlp(x_pad, W1, b1, W2, b2):
    return pl.pallas_call(
        _mlp_body,
        grid=(NP // MLP_BLK,),
        in_specs=[
            pl.BlockSpec((MLP_BLK, IN_C), lambda i: (i, 0)),
            pl.BlockSpec((IN_C, HID), lambda i: (0, 0)),
            pl.BlockSpec((1, HID), lambda i: (0, 0)),
            pl.BlockSpec((HID, OUT_C), lambda i: (0, 0)),
            pl.BlockSpec((1, OUT_C), lambda i: (0, 0)),
        ],
        out_specs=pl.BlockSpec((MLP_BLK, OUT_C), lambda i: (i, 0)),
        out_shape=jax.ShapeDtypeStruct((NP, OUT_C), jnp.float32),
    )(x_pad, W1, b1.reshape(1, HID), W2, b2.reshape(1, OUT_C))


CB = 128


def _final_body(*refs):
    pred_refs = refs[:K + 1]
    wp_ref, bp_ref, out_ref = refs[K + 1], refs[K + 2], refs[K + 3]
    acc = jnp.zeros((CB, OUT_C), jnp.float32)
    for pr in pred_refs:
        p = pr[...]
        r = jax.nn.sigmoid(
            jnp.dot(p, wp_ref[...], preferred_element_type=jnp.float32)
            + bp_ref[...])
        acc = acc + r * p
    out_ref[...] = acc


def _final(preds, Wp, bp2):
    return pl.pallas_call(
        _final_body,
        grid=(NP // CB,),
        in_specs=(
            [pl.BlockSpec((CB, OUT_C), lambda i: (i, 0)) for _ in range(K + 1)]
            + [pl.BlockSpec((OUT_C, 1), lambda i: (0, 0)),
               pl.BlockSpec((1, 1), lambda i: (0, 0))]
        ),
        out_specs=pl.BlockSpec((CB, OUT_C), lambda i: (i, 0)),
        out_shape=jax.ShapeDtypeStruct((NP, OUT_C), jnp.float32),
    )(*preds, Wp, bp2)


def _interleave(flat):
    return flat.reshape(HOP_B * 128, NT).T.reshape(NT, HOP_B, 128)


def kernel(x, edge_index, W1, b1, W2, b2, Wp, bp):
    row = edge_index[0]
    col = edge_index[1]
    loop = jnp.arange(N, dtype=jnp.int32)

    padD = jnp.full((ED - E,), N, dtype=jnp.int32)
    rowd = jnp.concatenate([row, padD]).reshape(NT, DEG_B * 128)
    cold = jnp.concatenate([col, padD]).reshape(NT, DEG_B * 128)

    rowp = jnp.where(row == col, N, row)
    padH = jnp.full((EH - E - N,), N, dtype=jnp.int32)
    rowh = jnp.concatenate([rowp, loop, padH])
    colh = jnp.concatenate([col, loop, padH])
    ehp = _interleave((rowh << 14) | colh)

    x_pad = jnp.concatenate([x, jnp.zeros((NP - N, IN_C), jnp.float32)])
    bp2 = bp.reshape(1, 1)

    h = _mlp(x_pad, W1, b1, W2, b2)
    part_deg = _deg_kernel(rowd, cold)
    dis, p = _prep_kernel(part_deg, h)
    preds = []
    for _ in range(K):
        p, pk = _hop_kernel(p, dis, ehp)
        preds.append(pk)
    preds.append(_merge_kernel(p, dis))
    out = _final(preds, Wp, bp2)
    return out[:N]


# speedup vs baseline: 3.8017x; 1.1009x over previous
import functools

import jax
import jax.numpy as jnp
from jax import lax
from jax.experimental import pallas as pl
from jax.experimental.pallas import tpu as pltpu
from jax.experimental.pallas import tpu_sc as plsc

N = 10000
NP = 10240
E = 320000
IN_C = 128
HID = 256
OUT_C = 64
K = 10

NT = 32
NSUB = 16
DEG_B = 79
HOP_B = 82
ED = NT * DEG_B * 128
EH = NT * HOP_B * 128
RPT = NP // NSUB
RPW = NP // NT
NBUF = 2
P1B = 32

_mesh = plsc.VectorSubcoreMesh(core_axis_name="c", subcore_axis_name="s")
_sc_params = pltpu.CompilerParams(needs_layout_passes=False,
                                  use_tc_tiling_on_sc=False)


def _rsqrt16(d):
    i = plsc.bitcast(d, jnp.int32)
    y = plsc.bitcast(jnp.int32(0x5F3759DF) - (i >> 1), jnp.float32)
    for _ in range(3):
        y = y * (1.5 - 0.5 * d * y * y)
    return y


@functools.partial(
    pl.kernel,
    mesh=_mesh,
    out_type=jax.ShapeDtypeStruct((NT, NP), jnp.float32),
    scratch_types=[
        pltpu.VMEM((DEG_B * 128,), jnp.int32),
        pltpu.VMEM((DEG_B * 128,), jnp.int32),
        pltpu.VMEM((NP,), jnp.float32),
    ],
    compiler_params=_sc_params,
)
def _deg_kernel(rowd, cold, part, rvm, cvm, acc):
    c = lax.axis_index("c")
    s = lax.axis_index("s")
    wid = c * NSUB + s
    pltpu.sync_copy(rowd.at[wid], rvm)
    pltpu.sync_copy(cold.at[wid], cvm)

    def _zero(i, carry):
        acc[pl.ds(i * 16, 16)] = jnp.zeros((16,), jnp.float32)
        return carry

    lax.fori_loop(0, NP // 16, _zero, 0)

    def _edge(i, carry):
        r = rvm[pl.ds(i * 16, 16)]
        cc = cvm[pl.ds(i * 16, 16)]
        ew = jnp.where(r != cc, 1.0, 0.0).astype(jnp.float32)
        plsc.addupdate_scatter(acc, [cc], ew)
        return carry

    lax.fori_loop(0, (DEG_B * 128) // 16, _edge, 0)
    pltpu.sync_copy(acc, part.at[wid])


@functools.partial(
    pl.kernel,
    mesh=_mesh,
    out_type=[
        jax.ShapeDtypeStruct((NP,), jnp.float32),
        jax.ShapeDtypeStruct((2, NP, OUT_C), jnp.float32),
    ],
    scratch_types=[
        pltpu.VMEM((NT, RPW), jnp.float32),
        pltpu.VMEM((RPW,), jnp.float32),
        pltpu.VMEM((RPW,), jnp.float32),
        pltpu.VMEM((RPW, OUT_C), jnp.float32),
        pltpu.VMEM((RPW, OUT_C), jnp.float32),
    ],
    compiler_params=_sc_params,
)
def _prep_kernel(part, h, dis_out, pinit, pbuf, disv, sdv, hv, wv):
    c = lax.axis_index("c")
    s = lax.axis_index("s")
    wid = c * NSUB + s
    base = wid * RPW
    for k in range(NT):
        pltpu.sync_copy(part.at[k, pl.ds(base, RPW)], pbuf.at[k])
    pltpu.sync_copy(h.at[pl.ds(base, RPW)], hv)

    def _vec(v, carry):
        d = pbuf[0, pl.ds(v * 16, 16)]
        for k in range(1, NT):
            d = d + pbuf[k, pl.ds(v * 16, 16)]
        d = d + 1.0
        r = _rsqrt16(d)
        disv[pl.ds(v * 16, 16)] = r
        sdv[pl.ds(v * 16, 16)] = d * r
        return carry

    lax.fori_loop(0, RPW // 16, _vec, 0)
    pltpu.sync_copy(disv, dis_out.at[pl.ds(base, RPW)])

    def _rowg(g, carry):
        s16 = sdv[pl.ds(g * 16, 16)]
        i16 = base + g * 16 + lax.iota(jnp.int32, 16)
        s16 = jnp.where(i16 < N, s16, 0.0)
        for r in range(16):
            i = g * 16 + r
            sc = s16[r]
            for l in range(OUT_C // 16):
                wv[i, pl.ds(l * 16, 16)] = hv[i, pl.ds(l * 16, 16)] * sc
        return carry

    lax.fori_loop(0, RPW // 16, _rowg, 0)
    pltpu.sync_copy(wv, pinit.at[0, pl.ds(base, RPW)])

    def _zrow(i, carry):
        for l in range(OUT_C // 16):
            wv[i, pl.ds(l * 16, 16)] = jnp.zeros((16,), jnp.float32)
        return carry

    lax.fori_loop(0, RPW, _zrow, 0)
    pltpu.sync_copy(wv, pinit.at[1, pl.ds(base, RPW)])


@functools.partial(
    pl.kernel,
    mesh=_mesh,
    out_type=[
        jax.ShapeDtypeStruct((2, NP, OUT_C), jnp.float32),
        jax.ShapeDtypeStruct((NP, OUT_C), jnp.float32),
    ],
    scratch_types=[
        pltpu.VMEM((HOP_B, 128), jnp.int32),
        pltpu.VMEM((HOP_B, 128), jnp.int32),
        pltpu.VMEM((RPT,), jnp.float32),
        [pltpu.VMEM((P1B, OUT_C), jnp.float32) for _ in range(2)],
        [pltpu.VMEM((P1B, OUT_C), jnp.float32) for _ in range(2)],
        pltpu.VMEM((P1B, OUT_C), jnp.float32),
        [pltpu.SemaphoreType.DMA for _ in range(2)],
        [pltpu.VMEM((128, OUT_C), jnp.float32) for _ in range(NBUF)],
        [pltpu.SemaphoreType.DMA for _ in range(NBUF)],
        [pltpu.SemaphoreType.DMA for _ in range(NBUF)],
        pltpu.VMEM_SHARED((NP, OUT_C), jnp.float32),
        pltpu.VMEM_SHARED((NP, OUT_C), jnp.float32),
    ],
    compiler_params=_sc_params,
)
def _hop_kernel(pprev, dis, eh, part, pred,
                rvm, cvm, disv, pa, pb, zb, sl, gb, sg, ss, acc, u_sp):
    c = lax.axis_index("c")
    s = lax.axis_index("s")
    wid = c * NSUB + s
    pltpu.sync_copy(eh.at[wid], rvm)
    base = s * RPT
    pltpu.sync_copy(dis.at[pl.ds(base, RPT)], disv)

    def _unpack(j, carry):
        for l in range(128 // 16):
            v = rvm[j, pl.ds(l * 16, 16)]
            cvm[j, pl.ds(l * 16, 16)] = lax.bitwise_and(v, 16383)
            rvm[j, pl.ds(l * 16, 16)] = lax.shift_right_logical(v, 14)
        return carry

    lax.fori_loop(0, HOP_B, _unpack, 0)

    def _zrow(i, carry):
        for l in range(OUT_C // 16):
            zb[i, pl.ds(l * 16, 16)] = jnp.zeros((16,), jnp.float32)
        return carry

    lax.fori_loop(0, P1B, _zrow, 0)

    NSC = RPT // P1B
    pltpu.async_copy(pprev.at[0, pl.ds(base, P1B)], pa[0], sl[0])
    pltpu.async_copy(pprev.at[1, pl.ds(base, P1B)], pb[0], sl[0])
    for sub in range(NSC):
        pr = sub % 2
        rb = base + sub * P1B
        pltpu.make_async_copy(
            pprev.at[0, pl.ds(rb, P1B)], pa[pr], sl[pr]).wait()
        pltpu.make_async_copy(
            pprev.at[1, pl.ds(rb, P1B)], pb[pr], sl[pr]).wait()
        if sub + 1 < NSC:
            nq = 1 - pr
            pltpu.async_copy(
                pprev.at[0, pl.ds(rb + P1B, P1B)], pa[nq], sl[nq])
            pltpu.async_copy(
                pprev.at[1, pl.ds(rb + P1B, P1B)], pb[nq], sl[nq])

        def _rowg(g, carry, _sub=sub, _pr=pr):
            d16 = disv[pl.ds(_sub * P1B + g * 16, 16)]
            for r in range(16):
                i = g * 16 + r
                d = d16[r]
                for l in range(OUT_C // 16):
                    t = (pa[_pr][i, pl.ds(l * 16, 16)]
                         + pb[_pr][i, pl.ds(l * 16, 16)]) * d
                    pa[_pr][i, pl.ds(l * 16, 16)] = t
                    pb[_pr][i, pl.ds(l * 16, 16)] = t * d
            return carry

        lax.fori_loop(0, P1B // 16, _rowg, 0)

        @pl.when(c == 0)
        def _(pr=pr, rb=rb):
            pltpu.sync_copy(pa[pr], pred.at[pl.ds(rb, P1B)])

        pltpu.sync_copy(pb[pr], u_sp.at[pl.ds(rb, P1B)])
        pltpu.sync_copy(zb, acc.at[pl.ds(rb, P1B)])

    plsc.subcore_barrier()

    for b in range(NBUF):
        pltpu.async_copy(u_sp.at[rvm.at[b]], gb[b], sg[b])

    def _ring(it, carry):
        for b in range(NBUF):
            j = it * NBUF + b
            pltpu.make_async_copy(u_sp.at[rvm.at[j]], gb[b], sg[b]).wait()
            pltpu.async_copy(gb[b], acc.at[cvm.at[j]], ss[b], add=True)
        for b in range(NBUF):
            nj = it * NBUF + b + NBUF

            @pl.when(nj < HOP_B)
            def _(b=b, nj=nj):
                pltpu.make_async_copy(gb[b], acc.at[cvm.at[nj]], ss[b]).wait()
                pltpu.async_copy(u_sp.at[rvm.at[nj]], gb[b], sg[b])
        return carry

    lax.fori_loop(0, HOP_B // NBUF, _ring, 0)
    for b in range(NBUF):
        pltpu.make_async_copy(gb[b], acc.at[cvm.at[0]], ss[b]).wait()

    plsc.subcore_barrier()
    pltpu.sync_copy(acc.at[pl.ds(base, RPT)], part.at[c, pl.ds(base, RPT)])


@functools.partial(
    pl.kernel,
    mesh=_mesh,
    out_type=jax.ShapeDtypeStruct((NP, OUT_C), jnp.float32),
    scratch_types=[
        pltpu.VMEM((RPW,), jnp.float32),
        pltpu.VMEM((RPW, OUT_C), jnp.float32),
        pltpu.VMEM((RPW, OUT_C), jnp.float32),
    ],
    compiler_params=_sc_params,
)
def _merge_kernel(pprev, dis, pred, disv, pa, pb):
    c = lax.axis_index("c")
    s = lax.axis_index("s")
    wid = c * NSUB + s
    base = wid * RPW
    pltpu.sync_copy(dis.at[pl.ds(base, RPW)], disv)
    pltpu.sync_copy(pprev.at[0, pl.ds(base, RPW)], pa)
    pltpu.sync_copy(pprev.at[1, pl.ds(base, RPW)], pb)

    def _rowg(g, carry):
        d16 = disv[pl.ds(g * 16, 16)]
        for r in range(16):
            i = g * 16 + r
            d = d16[r]
            for l in range(OUT_C // 16):
                pa[i, pl.ds(l * 16, 16)] = (
                    pa[i, pl.ds(l * 16, 16)] + pb[i, pl.ds(l * 16, 16)]) * d
        return carry

    lax.fori_loop(0, RPW // 16, _rowg, 0)
    pltpu.sync_copy(pa, pred.at[pl.ds(base, RPW)])


def _mlp_body(x_ref, w1_ref, b1_ref, w2_ref, b2_ref, h_ref):
    h1 = jnp.maximum(
        jnp.dot(x_ref[...], w1_ref[...], preferred_element_type=jnp.float32)
        + b1_ref[...], 0.0)
    h_ref[...] = (
        jnp.dot(h1, w2_ref[...], preferred_element_type=jnp.float32)
        + b2_ref[...])


MLP_BLK = 512


def _mlp(x_pad, W1, b1, W2, b2):
    return pl.pallas_call(
        _mlp_body,
        grid=(NP // MLP_BLK,),
        in_specs=[
            pl.BlockSpec((MLP_BLK, IN_C), lambda i: (i, 0)),
            pl.BlockSpec((IN_C, HID), lambda i: (0, 0)),
            pl.BlockSpec((1, HID), lambda i: (0, 0)),
            pl.BlockSpec((HID, OUT_C), lambda i: (0, 0)),
            pl.BlockSpec((1, OUT_C), lambda i: (0, 0)),
        ],
        out_specs=pl.BlockSpec((MLP_BLK, OUT_C), lambda i: (i, 0)),
        out_shape=jax.ShapeDtypeStruct((NP, OUT_C), jnp.float32),
    )(x_pad, W1, b1.reshape(1, HID), W2, b2.reshape(1, OUT_C))


CB = 128


def _final_body(*refs):
    pred_refs = refs[:K + 1]
    wp_ref, bp_ref, out_ref = refs[K + 1], refs[K + 2], refs[K + 3]
    acc = jnp.zeros((CB, OUT_C), jnp.float32)
    for pr in pred_refs:
        p = pr[...]
        r = jax.nn.sigmoid(
            jnp.dot(p, wp_ref[...], preferred_element_type=jnp.float32)
            + bp_ref[...])
        acc = acc + r * p
    out_ref[...] = acc


def _final(preds, Wp, bp2):
    return pl.pallas_call(
        _final_body,
        grid=(NP // CB,),
        in_specs=(
            [pl.BlockSpec((CB, OUT_C), lambda i: (i, 0)) for _ in range(K + 1)]
            + [pl.BlockSpec((OUT_C, 1), lambda i: (0, 0)),
               pl.BlockSpec((1, 1), lambda i: (0, 0))]
        ),
        out_specs=pl.BlockSpec((CB, OUT_C), lambda i: (i, 0)),
        out_shape=jax.ShapeDtypeStruct((NP, OUT_C), jnp.float32),
    )(*preds, Wp, bp2)


def _interleave(flat):
    return flat.reshape(HOP_B * 128, NT).T.reshape(NT, HOP_B, 128)


def kernel(x, edge_index, W1, b1, W2, b2, Wp, bp):
    row = edge_index[0]
    col = edge_index[1]
    loop = jnp.arange(N, dtype=jnp.int32)

    padD = jnp.full((ED - E,), N, dtype=jnp.int32)
    rowd = jnp.concatenate([row, padD]).reshape(NT, DEG_B * 128)
    cold = jnp.concatenate([col, padD]).reshape(NT, DEG_B * 128)

    rowp = jnp.where(row == col, N, row)
    padH = jnp.full((EH - E - N,), N, dtype=jnp.int32)
    rowh = jnp.concatenate([rowp, loop, padH])
    colh = jnp.concatenate([col, loop, padH])
    ehp = _interleave((rowh << 14) | colh)

    x_pad = jnp.concatenate([x, jnp.zeros((NP - N, IN_C), jnp.float32)])
    bp2 = bp.reshape(1, 1)

    h = _mlp(x_pad, W1, b1, W2, b2)
    part_deg = _deg_kernel(rowd, cold)
    dis, p = _prep_kernel(part_deg, h)
    preds = []
    for _ in range(K):
        p, pk = _hop_kernel(p, dis, ehp)
        preds.append(pk)
    preds.append(_merge_kernel(p, dis))
    out = _final(preds, Wp, bp2)
    return out[:N]
